# Initial kernel scaffold; baseline (speedup 1.0000x reference)
import jax
import jax.numpy as jnp
from jax.experimental import pallas as pl


def kernel(uncertainty, embeddings, labels, epoch, max_epochs):
    raise NotImplementedError("write your pallas kernel here")


# speedup vs baseline: 2.4960x; 2.4960x over previous
import functools

import jax
import jax.numpy as jnp
from jax.experimental import pallas as pl
from jax.experimental.pallas import tpu as pltpu

N = 65536
D = 256
BLK = 4096
NB = N // BLK
RROWS = N // 128
TROWS = BLK // 128
K_FORCE = max(1, int(N * 0.1))
K_CAP = int(N * 0.5)


def _body(thr_ref, u0_ref, u1_ref, u2_ref, lab_ref, emb_ref,
          ms_ref, dist_ref,
          avg_s, acc_s, cnt_s, cen_s, c2_s):
    p = pl.program_id(0)
    j = pl.program_id(1)

    @pl.when((p == 0) & (j == 0))
    def _init():
        acc_s[...] = jnp.zeros_like(acc_s)
        cnt_s[...] = jnp.zeros_like(cnt_s)

    @pl.when(p == 0)
    def _phase0():
        tile = (u0_ref[...] + u1_ref[...] + u2_ref[...]) / 3.0
        avg_s[pl.ds(j * TROWS, TROWS), :] = tile
        lab = lab_ref[...]
        oh = (lab == jax.lax.broadcasted_iota(jnp.int32, (BLK, 8), 1))
        ohf = oh.astype(jnp.float32)
        acc_s[...] += jax.lax.dot_general(
            emb_ref[...], ohf, (((0,), (0,)), ((), ())),
            preferred_element_type=jnp.float32,
            precision=jax.lax.Precision.HIGHEST)
        cnt_s[...] += jnp.sum(ohf, axis=0, keepdims=True)
        dist_ref[...] = jnp.zeros_like(dist_ref)

    @pl.when((p == 0) & (j == NB - 1))
    def _finalize():
        invc = 1.0 / jnp.maximum(cnt_s[...], 1.0)
        cen = acc_s[...] * invc
        cen_s[...] = cen
        c2_s[...] = jnp.sum(cen * cen, axis=0, keepdims=True)

        avg = avg_s[...]
        thr = thr_ref[0]
        keys = jax.lax.bitcast_convert_type(avg, jnp.int32)
        hard_count = jnp.sum((avg > thr).astype(jnp.int32))
        k_eff = jnp.where(hard_count == 0, K_FORCE, K_CAP)

        def bs(_, carry):
            lo, hi = carry
            mid = jax.lax.shift_right_arithmetic(lo + hi, 1)
            cnt = jnp.sum((keys > mid).astype(jnp.int32))
            pred = cnt >= k_eff
            return (jnp.where(pred, mid, lo), jnp.where(pred, hi, mid))

        lo, hi = jax.lax.fori_loop(
            0, 31, bs, (jnp.int32(-1), jnp.int32(0x40000000)))
        use_thr = (hard_count > 0) & (hard_count <= K_CAP)
        sel = jnp.where(use_thr, (avg > thr).astype(jnp.float32),
                        (keys >= hi).astype(jnp.float32))
        ms_ref[...] = avg * sel

    @pl.when(p == 1)
    def _phase1():
        e = emb_ref[...]
        prod = jax.lax.dot_general(
            e, cen_s[...], (((1,), (0,)), ((), ())),
            preferred_element_type=jnp.float32,
            precision=jax.lax.Precision.HIGHEST)
        x2 = jnp.sum(e * e, axis=1, keepdims=True)
        d2 = jnp.maximum(x2 + c2_s[...] - 2.0 * prod, 0.0)
        dist_ref[...] = jnp.sqrt(d2 + 1e-12)


@functools.partial(jax.jit, static_argnames=("interpret",))
def _run(thr, u0, u1, u2, lab_col, emb, interpret=False):
    ms, dist8 = pl.pallas_call(
        _body,
        grid=(2, NB),
        in_specs=[
            pl.BlockSpec(memory_space=pltpu.SMEM),
            pl.BlockSpec((TROWS, 128), lambda p, j: (j, 0)),
            pl.BlockSpec((TROWS, 128), lambda p, j: (j, 0)),
            pl.BlockSpec((TROWS, 128), lambda p, j: (j, 0)),
            pl.BlockSpec((BLK, 1), lambda p, j: (j, 0)),
            pl.BlockSpec((BLK, D), lambda p, j: (j, 0)),
        ],
        out_specs=[
            pl.BlockSpec((RROWS, 128), lambda p, j: (0, 0)),
            pl.BlockSpec((BLK, 8), lambda p, j: (j, 0)),
        ],
        out_shape=[
            jax.ShapeDtypeStruct((RROWS, 128), jnp.float32),
            jax.ShapeDtypeStruct((N, 8), jnp.float32),
        ],
        scratch_shapes=[
            pltpu.VMEM((RROWS, 128), jnp.float32),
            pltpu.VMEM((D, 8), jnp.float32),
            pltpu.VMEM((1, 8), jnp.float32),
            pltpu.VMEM((D, 8), jnp.float32),
            pltpu.VMEM((1, 8), jnp.float32),
        ],
        compiler_params=pltpu.CompilerParams(
            dimension_semantics=("arbitrary", "arbitrary")),
        interpret=interpret,
    )(thr, u0, u1, u2, lab_col, emb)
    return ms, dist8


def kernel(uncertainty, embeddings, labels, epoch, max_epochs):
    progress = jnp.minimum(epoch / jnp.maximum(max_epochs - 1, 1), 1.0)
    u_thr = 0.4 + progress * (0.3 - 0.4)
    c_thr = 0.3 + progress * (0.6 - 0.3)
    thr = jnp.minimum(u_thr, -jnp.log(c_thr)).astype(jnp.float32)
    thr = jnp.reshape(thr, (1,))

    u0 = uncertainty[:, 0].reshape(RROWS, 128)
    u1 = uncertainty[:, 1].reshape(RROWS, 128)
    u2 = uncertainty[:, 2].reshape(RROWS, 128)
    lab_col = labels.reshape(N, 1)

    ms, dist8 = _run(thr, u0, u1, u2, lab_col, embeddings)
    return jnp.concatenate([ms.reshape(N, 1), dist8[:, :3]], axis=1)


# speedup vs baseline: 3.3093x; 1.3259x over previous
import functools

import jax
import jax.numpy as jnp
from jax.experimental import pallas as pl
from jax.experimental.pallas import tpu as pltpu

N = 65536
D = 256
BLK = 4096
NB = N // BLK
RROWS = N // 128
TROWS = BLK // 128
K_FORCE = max(1, int(N * 0.1))
K_CAP = int(N * 0.5)


def _body(thr_ref, u0_ref, u1_ref, u2_ref, lab_ref, emb_ref,
          ms_ref, dist_ref,
          avg_s, acc_s, cnt_s, cen_s, c2_s):
    p = pl.program_id(0)
    j = pl.program_id(1)

    @pl.when((p == 0) & (j == 0))
    def _init():
        acc_s[...] = jnp.zeros_like(acc_s)
        cnt_s[...] = jnp.zeros_like(cnt_s)

    @pl.when(p == 0)
    def _phase0():
        tile = (u0_ref[...] + u1_ref[...] + u2_ref[...]) / 3.0
        avg_s[pl.ds(j * TROWS, TROWS), :] = tile
        lab = lab_ref[...]
        oh = (lab == jax.lax.broadcasted_iota(jnp.int32, (BLK, 8), 1))
        ohf = oh.astype(jnp.float32)
        acc_s[...] += jax.lax.dot_general(
            ohf, emb_ref[...], (((0,), (0,)), ((), ())),
            preferred_element_type=jnp.float32)
        cnt_s[...] += jnp.sum(ohf, axis=0, keepdims=True)
        dist_ref[...] = jnp.zeros_like(dist_ref)

    @pl.when((p == 0) & (j == NB - 1))
    def _finalize():
        invc = 1.0 / jnp.maximum(cnt_s[...], 1.0)
        cen = jnp.transpose(acc_s[...]) * invc
        cen_s[...] = cen
        c2_s[...] = jnp.sum(cen * cen, axis=0, keepdims=True)

        avg = avg_s[...]
        thr = thr_ref[0]
        keys = jax.lax.bitcast_convert_type(avg, jnp.int32)
        hard_count = jnp.sum((avg > thr).astype(jnp.int32))
        k_eff = jnp.where(hard_count == 0, K_FORCE, K_CAP)

        def bs(_, carry):
            lo, hi = carry
            mid = jax.lax.shift_right_arithmetic(lo + hi, 1)
            cnt = jnp.sum((keys > mid).astype(jnp.int32))
            pred = cnt >= k_eff
            return (jnp.where(pred, mid, lo), jnp.where(pred, hi, mid))

        lo, hi = jax.lax.fori_loop(
            0, 31, bs, (jnp.int32(-1), jnp.int32(0x40000000)))
        use_thr = (hard_count > 0) & (hard_count <= K_CAP)
        sel = jnp.where(use_thr, (avg > thr).astype(jnp.float32),
                        (keys >= hi).astype(jnp.float32))
        ms_ref[...] = avg * sel

    @pl.when(p == 1)
    def _phase1():
        e = emb_ref[...]
        prod = jax.lax.dot_general(
            e, cen_s[...], (((1,), (0,)), ((), ())),
            preferred_element_type=jnp.float32)
        x2 = jnp.sum(e * e, axis=1, keepdims=True)
        d2 = jnp.maximum(x2 + c2_s[...] - 2.0 * prod, 0.0)
        dist_ref[...] = jnp.sqrt(d2 + 1e-12)


@functools.partial(jax.jit, static_argnames=("interpret",))
def _run(thr, u0, u1, u2, lab_col, emb, interpret=False):
    ms, dist8 = pl.pallas_call(
        _body,
        grid=(2, NB),
        in_specs=[
            pl.BlockSpec(memory_space=pltpu.SMEM),
            pl.BlockSpec((TROWS, 128), lambda p, j: (j, 0)),
            pl.BlockSpec((TROWS, 128), lambda p, j: (j, 0)),
            pl.BlockSpec((TROWS, 128), lambda p, j: (j, 0)),
            pl.BlockSpec((BLK, 1), lambda p, j: (j, 0)),
            pl.BlockSpec((BLK, D), lambda p, j: (j, 0)),
        ],
        out_specs=[
            pl.BlockSpec((RROWS, 128), lambda p, j: (0, 0)),
            pl.BlockSpec((BLK, 8), lambda p, j: (j, 0)),
        ],
        out_shape=[
            jax.ShapeDtypeStruct((RROWS, 128), jnp.float32),
            jax.ShapeDtypeStruct((N, 8), jnp.float32),
        ],
        scratch_shapes=[
            pltpu.VMEM((RROWS, 128), jnp.float32),
            pltpu.VMEM((8, D), jnp.float32),
            pltpu.VMEM((1, 8), jnp.float32),
            pltpu.VMEM((D, 8), jnp.float32),
            pltpu.VMEM((1, 8), jnp.float32),
        ],
        compiler_params=pltpu.CompilerParams(
            dimension_semantics=("arbitrary", "arbitrary")),
        interpret=interpret,
    )(thr, u0, u1, u2, lab_col, emb)
    return ms, dist8


def kernel(uncertainty, embeddings, labels, epoch, max_epochs):
    progress = jnp.minimum(epoch / jnp.maximum(max_epochs - 1, 1), 1.0)
    u_thr = 0.4 + progress * (0.3 - 0.4)
    c_thr = 0.3 + progress * (0.6 - 0.3)
    thr = jnp.minimum(u_thr, -jnp.log(c_thr)).astype(jnp.float32)
    thr = jnp.reshape(thr, (1,))

    u0 = uncertainty[:, 0].reshape(RROWS, 128)
    u1 = uncertainty[:, 1].reshape(RROWS, 128)
    u2 = uncertainty[:, 2].reshape(RROWS, 128)
    lab_col = labels.reshape(N, 1)

    ms, dist8 = _run(thr, u0, u1, u2, lab_col, embeddings)
    return jnp.concatenate([ms.reshape(N, 1), dist8[:, :3]], axis=1)


# speedup vs baseline: 5.1913x; 1.5687x over previous
import functools

import jax
import jax.numpy as jnp
from jax.experimental import pallas as pl
from jax.experimental.pallas import tpu as pltpu

N = 65536
D = 256
BLK = 8192
NB = N // BLK
RROWS = N // 128
TROWS = BLK // 128
K_FORCE = max(1, int(N * 0.1))
K_CAP = int(N * 0.5)


def _body(thr_ref, u0_ref, u1_ref, u2_ref, lab_ref, emb_ref,
          ms_ref, dist_ref,
          avg_s, acc_s, cnt_s, cen_s, c2_s):
    p = pl.program_id(0)
    j = pl.program_id(1)

    @pl.when((p == 0) & (j == 0))
    def _init():
        acc_s[...] = jnp.zeros_like(acc_s)
        cnt_s[...] = jnp.zeros_like(cnt_s)

    @pl.when(p == 0)
    def _phase0():
        tile = (u0_ref[...] + u1_ref[...] + u2_ref[...]) / 3.0
        avg_s[pl.ds(j * TROWS, TROWS), :] = tile
        lab = lab_ref[0]
        ohT = (lab == jax.lax.broadcasted_iota(jnp.int32, (8, BLK), 0))
        ohfT = ohT.astype(jnp.float32)
        acc_s[...] += jax.lax.dot_general(
            ohfT, emb_ref[...], (((1,), (0,)), ((), ())),
            preferred_element_type=jnp.float32)
        cnt_s[...] += jnp.sum(ohfT, axis=1, keepdims=True)

    @pl.when((p == 0) & (j == NB - 1))
    def _finalize():
        invc = 1.0 / jnp.maximum(jnp.transpose(cnt_s[...]), 1.0)
        cen = jnp.transpose(acc_s[...]) * invc
        cen_s[...] = cen
        c2_s[...] = jnp.sum(cen * cen, axis=0, keepdims=True)

        avg = avg_s[...]
        thr = thr_ref[0]
        keys = jax.lax.bitcast_convert_type(avg, jnp.int32)
        hard_count = jnp.sum((avg > thr).astype(jnp.int32))
        k_eff = jnp.where(hard_count == 0, K_FORCE, K_CAP)

        def bs(_, carry):
            lo, hi = carry
            mid = jax.lax.shift_right_arithmetic(lo + hi, 1)
            cnt = jnp.sum((keys > mid).astype(jnp.int32))
            pred = cnt >= k_eff
            return (jnp.where(pred, mid, lo), jnp.where(pred, hi, mid))

        lo, hi = jax.lax.fori_loop(
            0, 31, bs, (jnp.int32(-1), jnp.int32(0x40000000)))
        use_thr = (hard_count > 0) & (hard_count <= K_CAP)
        sel = jnp.where(use_thr, (avg > thr).astype(jnp.float32),
                        (keys >= hi).astype(jnp.float32))
        ms_ref[...] = avg * sel

    @pl.when(p == 1)
    def _phase1():
        e = emb_ref[...]
        prod = jax.lax.dot_general(
            e, cen_s[...], (((1,), (0,)), ((), ())),
            preferred_element_type=jnp.float32)
        x2 = jnp.sum(e * e, axis=1, keepdims=True)
        d2 = jnp.maximum(x2 + c2_s[...] - 2.0 * prod, 0.0)
        dist_ref[...] = jnp.sqrt(d2 + 1e-12)


def _pin0(p, j):
    return jnp.where(p == 0, j, 0)


def _body_specs():
    return dict(
        grid=(2, NB),
        in_specs=[
            pl.BlockSpec(memory_space=pltpu.SMEM),
            pl.BlockSpec((TROWS, 128), lambda p, j: (_pin0(p, j), 0)),
            pl.BlockSpec((TROWS, 128), lambda p, j: (_pin0(p, j), 0)),
            pl.BlockSpec((TROWS, 128), lambda p, j: (_pin0(p, j), 0)),
            pl.BlockSpec((1, 1, BLK), lambda p, j: (_pin0(p, j), 0, 0)),
            pl.BlockSpec((BLK, D), lambda p, j: (j, 0)),
        ],
        out_specs=[
            pl.BlockSpec((RROWS, 128), lambda p, j: (0, 0)),
            pl.BlockSpec((BLK, 8), lambda p, j: (jnp.where(p == 0, 0, j), 0)),
        ],
        out_shape=[
            jax.ShapeDtypeStruct((RROWS, 128), jnp.float32),
            jax.ShapeDtypeStruct((N, 8), jnp.float32),
        ],
        scratch_shapes=[
            pltpu.VMEM((RROWS, 128), jnp.float32),
            pltpu.VMEM((8, D), jnp.float32),
            pltpu.VMEM((8, 1), jnp.float32),
            pltpu.VMEM((D, 8), jnp.float32),
            pltpu.VMEM((1, 8), jnp.float32),
        ],
        compiler_params=pltpu.CompilerParams(
            dimension_semantics=("arbitrary", "arbitrary")),
    )


@functools.partial(jax.jit, static_argnames=("interpret",))
def _run(thr, u0, u1, u2, lab3, emb, interpret=False):
    ms, dist8 = pl.pallas_call(
        _body, **_body_specs(), interpret=interpret,
    )(thr, u0, u1, u2, lab3, emb)
    return ms, dist8


def kernel(uncertainty, embeddings, labels, epoch, max_epochs):
    progress = jnp.minimum(epoch / jnp.maximum(max_epochs - 1, 1), 1.0)
    u_thr = 0.4 + progress * (0.3 - 0.4)
    c_thr = 0.3 + progress * (0.6 - 0.3)
    thr = jnp.minimum(u_thr, -jnp.log(c_thr)).astype(jnp.float32)
    thr = jnp.reshape(thr, (1,))

    u0 = uncertainty[:, 0].reshape(RROWS, 128)
    u1 = uncertainty[:, 1].reshape(RROWS, 128)
    u2 = uncertainty[:, 2].reshape(RROWS, 128)
    lab3 = labels.reshape(NB, 1, BLK)

    ms, dist8 = _run(thr, u0, u1, u2, lab3, embeddings)
    return jnp.concatenate([ms.reshape(N, 1), dist8[:, :3]], axis=1)
